# baseline (speedup 1.0000x reference)
import functools

import jax
import jax.numpy as jnp
from jax import lax
from jax.experimental import pallas as pl
from jax.experimental.pallas import tpu as pltpu
from jax.experimental.pallas import tpu_sc as plsc

D_MODEL = 2048
ID_DIM = 4096
W_DIM = D_MODEL * D_MODEL

NC = 2
NS = 16
L = 16

NUM_CHUNKS = 4
R_CHUNK = W_DIM // NUM_CHUNKS
BE = 4096
TRASH = 4096


def _sc_scatter_build(nnz_w, eb):
    per_tile_w = nnz_w // NS
    n_blocks = per_tile_w // BE
    n_pairs = n_blocks // 2
    stripe = R_CHUNK // NS
    eb2 = eb // 2
    mesh = plsc.VectorSubcoreMesh(core_axis_name="c", subcore_axis_name="s",
                                  num_cores=NC, num_subcores=NS)

    @functools.partial(
        pl.kernel,
        out_type=[
            jax.ShapeDtypeStruct((W_DIM,), jnp.float32),
            jax.ShapeDtypeStruct((NC, D_MODEL), jnp.float32),
        ],
        mesh=mesh,
        compiler_params=pltpu.CompilerParams(needs_layout_passes=False),
        scratch_types=[
            pltpu.VMEM((ID_DIM,), jnp.float32),
            pltpu.VMEM((BE,), jnp.int32),
            pltpu.VMEM((BE,), jnp.int32),
            pltpu.VMEM((BE,), jnp.float32),
            pltpu.VMEM((BE,), jnp.int32),
            pltpu.VMEM((BE,), jnp.int32),
            pltpu.VMEM((BE,), jnp.float32),
            pltpu.VMEM((BE,), jnp.int32),
            pltpu.VMEM((BE,), jnp.float32),
            pltpu.VMEM((BE,), jnp.int32),
            pltpu.VMEM((BE,), jnp.float32),
            pltpu.VMEM((eb2,), jnp.int32),
            pltpu.VMEM((eb2,), jnp.int32),
            pltpu.VMEM((eb2,), jnp.float32),
            pltpu.VMEM((eb2,), jnp.int32),
            pltpu.VMEM((eb2,), jnp.float32),
            pltpu.VMEM_SHARED((R_CHUNK + TRASH,), jnp.float32),
            pltpu.VMEM_SHARED((D_MODEL,), jnp.float32),
            pltpu.SemaphoreType.DMA,
            pltpu.SemaphoreType.DMA,
            pltpu.SemaphoreType.DMA,
            pltpu.SemaphoreType.DMA,
        ],
    )
    def sc_kernel(v_hbm, w0_hbm, rows_hbm, cols_hbm, vals_hbm,
                  brows_hbm, bcols_hbm, bvals_hbm, zeros_hbm,
                  w_out, pb_out,
                  v_v, rows0_v, cols0_v, vals0_v, rows1_v, cols1_v, vals1_v,
                  idx0_v, val0_v, idx1_v, val1_v,
                  brows_v, bcols_v, bvals_v, bidx_v, bval_v,
                  wacc_s, bacc_s,
                  in_sem0, in_sem1, sc_sem0, sc_sem1):
        cid = lax.axis_index("c")
        sid = lax.axis_index("s")
        insets = ((rows0_v, cols0_v, vals0_v, in_sem0),
                  (rows1_v, cols1_v, vals1_v, in_sem1))
        stsets = ((idx0_v, val0_v, sc_sem0), (idx1_v, val1_v, sc_sem1))

        pltpu.sync_copy(v_hbm, v_v)

        @pl.when(sid == 0)
        def _():
            pltpu.sync_copy(zeros_hbm, bacc_s)

        def start_in(blk, s):
            rb, cb, vb, sem = insets[s]
            estart = sid * per_tile_w + blk * BE
            pltpu.async_copy(rows_hbm.at[pl.ds(estart, BE)], rb, sem)
            pltpu.async_copy(cols_hbm.at[pl.ds(estart, BE)], cb, sem)
            pltpu.async_copy(vals_hbm.at[pl.ds(estart, BE)], vb, sem)

        def wait_in(blk, s):
            rb, cb, vb, sem = insets[s]
            estart = sid * per_tile_w + blk * BE
            pltpu.make_async_copy(rows_hbm.at[pl.ds(estart, BE)], rb, sem).wait()
            pltpu.make_async_copy(cols_hbm.at[pl.ds(estart, BE)], cb, sem).wait()
            pltpu.make_async_copy(vals_hbm.at[pl.ds(estart, BE)], vb, sem).wait()

        def scan_chunk(base):
            start_in(0, 0)
            start_in(1, 1)

            def pair(p, carry):
                for s in range(2):
                    blk = 2 * p + s
                    rb, cb, vb, _ = insets[s]
                    ib, wb, ssem = stsets[s]
                    wait_in(blk, s)

                    @pl.when(p >= 1)
                    def _():
                        pltpu.make_async_copy(wb, wacc_s.at[ib], ssem).wait()

                    @plsc.parallel_loop(0, BE, L, unroll=16)
                    def _(off):
                        r16 = rb[pl.ds(off, L)]
                        c16 = cb[pl.ds(off, L)]
                        a16 = vb[pl.ds(off, L)]
                        vv = plsc.load_gather(v_v, [c16])
                        li = r16 - base
                        inb = (li >= 0) & (li < R_CHUNK)
                        trash = R_CHUNK + (r16 & (TRASH - 1))
                        ib[pl.ds(off, L)] = jnp.where(inb, li, trash)
                        wb[pl.ds(off, L)] = a16 * vv

                    pltpu.async_copy(wb, wacc_s.at[ib], ssem, add=True)

                    @pl.when(p < n_pairs - 1)
                    def _():
                        start_in(blk + 2, s)
                return carry

            lax.fori_loop(0, n_pairs, pair, 0)
            for s in range(2):
                ib, wb, ssem = stsets[s]
                pltpu.make_async_copy(wb, wacc_s.at[ib], ssem).wait()

        for chunk_i in range(NUM_CHUNKS // NC):
            base = (cid * (NUM_CHUNKS // NC) + chunk_i) * R_CHUNK
            pltpu.sync_copy(w0_hbm.at[pl.ds(base + sid * stripe, stripe)],
                            wacc_s.at[pl.ds(sid * stripe, stripe)])
            plsc.subcore_barrier()
            scan_chunk(base)
            plsc.subcore_barrier()
            pltpu.sync_copy(wacc_s.at[pl.ds(sid * stripe, stripe)],
                            w_out.at[pl.ds(base + sid * stripe, stripe)])
            plsc.subcore_barrier()

        wid = sid * NC + cid
        for q in range(2):
            bstart = wid * eb + q * eb2
            pltpu.sync_copy(brows_hbm.at[pl.ds(bstart, eb2)], brows_v)
            pltpu.sync_copy(bcols_hbm.at[pl.ds(bstart, eb2)], bcols_v)
            pltpu.sync_copy(bvals_hbm.at[pl.ds(bstart, eb2)], bvals_v)

            def bgrp(g, c):
                off = g * L
                r16 = brows_v[pl.ds(off, L)]
                c16 = bcols_v[pl.ds(off, L)]
                a16 = bvals_v[pl.ds(off, L)]
                vv = plsc.load_gather(v_v, [c16])
                bidx_v[pl.ds(off, L)] = r16
                bval_v[pl.ds(off, L)] = a16 * vv
                return c

            lax.fori_loop(0, eb2 // L, bgrp, 0)
            pltpu.sync_copy(bval_v, bacc_s.at[bidx_v], add=True)
        plsc.subcore_barrier()

        @pl.when(sid == 0)
        def _():
            pltpu.sync_copy(bacc_s, pb_out.at[cid])

    return sc_kernel


def _mm_block(x_ref, w_ref, b0_ref, pb_ref, o_ref):
    b = b0_ref[0] + pb_ref[0] + pb_ref[1]
    o_ref[...] = lax.dot_general(
        x_ref[...], w_ref[...], (((1,), (1,)), ((), ())),
        preferred_element_type=jnp.float32) + b[None, :]


def _matmul(x, w, b0, pb):
    n_tok, d = x.shape
    bm, bn = 1024, 1024
    grid = (d // bn, n_tok // bm)
    return pl.pallas_call(
        _mm_block,
        grid=grid,
        in_specs=[
            pl.BlockSpec((bm, d), lambda j, i: (i, 0)),
            pl.BlockSpec((bn, d), lambda j, i: (j, 0)),
            pl.BlockSpec((1, bn), lambda j, i: (0, j)),
            pl.BlockSpec((NC, bn), lambda j, i: (0, j)),
        ],
        out_specs=pl.BlockSpec((bm, bn), lambda j, i: (i, j)),
        out_shape=jax.ShapeDtypeStruct((n_tok, d), jnp.float32),
    )(x, w, b0, pb)


def kernel(x, V, W0, b0, rows_w, cols_w, vals_w, rows_b, cols_b, vals_b):
    v = V[:, 0]
    nnz_w = rows_w.shape[0]
    nnz_b = rows_b.shape[0]

    nw = NC * NS
    eb = -(-nnz_b // (nw * 2 * L)) * 2 * L
    pad = nw * eb - nnz_b
    rbp = jnp.concatenate([rows_b, jnp.zeros((pad,), rows_b.dtype)])
    cbp = jnp.concatenate([cols_b, jnp.zeros((pad,), cols_b.dtype)])
    vbp = jnp.concatenate([vals_b, jnp.zeros((pad,), vals_b.dtype)])

    sc = _sc_scatter_build(nnz_w, eb)
    w_full, pb = sc(v, W0.reshape(-1), rows_w, cols_w, vals_w,
                    rbp, cbp, vbp, jnp.zeros((D_MODEL,), jnp.float32))
    return _matmul(x, w_full.reshape(D_MODEL, D_MODEL), b0.reshape(1, -1), pb)


# speedup vs baseline: 1.0014x; 1.0014x over previous
import functools

import jax
import jax.numpy as jnp
from jax import lax
from jax.experimental import pallas as pl
from jax.experimental.pallas import tpu as pltpu
from jax.experimental.pallas import tpu_sc as plsc

D_MODEL = 2048
ID_DIM = 4096
W_DIM = D_MODEL * D_MODEL

NC = 2
NS = 16
L = 16

NUM_CHUNKS = 4
R_CHUNK = W_DIM // NUM_CHUNKS
BE = 4096
TRASH = 4096


def _sc_scatter_build(nnz_w, eb):
    per_tile_w = nnz_w // NS
    n_blocks = per_tile_w // BE
    n_pairs = n_blocks // 2
    stripe = R_CHUNK // NS
    eb2 = eb // 2
    mesh = plsc.VectorSubcoreMesh(core_axis_name="c", subcore_axis_name="s",
                                  num_cores=NC, num_subcores=NS)

    @functools.partial(
        pl.kernel,
        out_type=[
            jax.ShapeDtypeStruct((W_DIM,), jnp.float32),
            jax.ShapeDtypeStruct((NC, D_MODEL), jnp.float32),
        ],
        mesh=mesh,
        compiler_params=pltpu.CompilerParams(needs_layout_passes=False),
        scratch_types=[
            pltpu.VMEM((ID_DIM,), jnp.float32),
            pltpu.VMEM((BE,), jnp.int32),
            pltpu.VMEM((BE,), jnp.int32),
            pltpu.VMEM((BE,), jnp.float32),
            pltpu.VMEM((BE,), jnp.int32),
            pltpu.VMEM((BE,), jnp.int32),
            pltpu.VMEM((BE,), jnp.float32),
            pltpu.VMEM((BE,), jnp.int32),
            pltpu.VMEM((BE,), jnp.float32),
            pltpu.VMEM((BE,), jnp.int32),
            pltpu.VMEM((BE,), jnp.float32),
            pltpu.VMEM((eb2,), jnp.int32),
            pltpu.VMEM((eb2,), jnp.int32),
            pltpu.VMEM((eb2,), jnp.float32),
            pltpu.VMEM((eb2,), jnp.int32),
            pltpu.VMEM((eb2,), jnp.float32),
            pltpu.VMEM_SHARED((R_CHUNK + TRASH,), jnp.float32),
            pltpu.VMEM_SHARED((D_MODEL,), jnp.float32),
            pltpu.SemaphoreType.DMA,
            pltpu.SemaphoreType.DMA,
            pltpu.SemaphoreType.DMA,
            pltpu.SemaphoreType.DMA,
        ],
    )
    def sc_kernel(v_hbm, w0_hbm, rows_hbm, cols_hbm, vals_hbm,
                  brows_hbm, bcols_hbm, bvals_hbm, zeros_hbm,
                  w_out, pb_out,
                  v_v, rows0_v, cols0_v, vals0_v, rows1_v, cols1_v, vals1_v,
                  idx0_v, val0_v, idx1_v, val1_v,
                  brows_v, bcols_v, bvals_v, bidx_v, bval_v,
                  wacc_s, bacc_s,
                  in_sem0, in_sem1, sc_sem0, sc_sem1):
        cid = lax.axis_index("c")
        sid = lax.axis_index("s")
        insets = ((rows0_v, cols0_v, vals0_v, in_sem0),
                  (rows1_v, cols1_v, vals1_v, in_sem1))
        stsets = ((idx0_v, val0_v, sc_sem0), (idx1_v, val1_v, sc_sem1))

        pltpu.sync_copy(v_hbm, v_v)

        @pl.when(sid == 0)
        def _():
            pltpu.sync_copy(zeros_hbm, bacc_s)

        def start_in(blk, s):
            rb, cb, vb, sem = insets[s]
            estart = sid * per_tile_w + blk * BE
            pltpu.async_copy(rows_hbm.at[pl.ds(estart, BE)], rb, sem)
            pltpu.async_copy(cols_hbm.at[pl.ds(estart, BE)], cb, sem)
            pltpu.async_copy(vals_hbm.at[pl.ds(estart, BE)], vb, sem)

        def wait_in(blk, s):
            rb, cb, vb, sem = insets[s]
            estart = sid * per_tile_w + blk * BE
            pltpu.make_async_copy(rows_hbm.at[pl.ds(estart, BE)], rb, sem).wait()
            pltpu.make_async_copy(cols_hbm.at[pl.ds(estart, BE)], cb, sem).wait()
            pltpu.make_async_copy(vals_hbm.at[pl.ds(estart, BE)], vb, sem).wait()

        def scan_chunk(base):
            start_in(0, 0)
            start_in(1, 1)

            def pair(p, carry):
                for s in range(2):
                    blk = 2 * p + s
                    rb, cb, vb, _ = insets[s]
                    ib, wb, ssem = stsets[s]
                    wait_in(blk, s)

                    @pl.when(p >= 1)
                    def _():
                        pltpu.make_async_copy(wb, wacc_s.at[ib], ssem).wait()

                    @plsc.parallel_loop(0, BE, L, unroll=8)
                    def _(off):
                        r16 = rb[pl.ds(off, L)]
                        c16 = cb[pl.ds(off, L)]
                        a16 = vb[pl.ds(off, L)]
                        vv = plsc.load_gather(v_v, [c16])
                        li = r16 - base
                        inb = (li >= 0) & (li < R_CHUNK)
                        trash = R_CHUNK + (r16 & (TRASH - 1))
                        ib[pl.ds(off, L)] = jnp.where(inb, li, trash)
                        wb[pl.ds(off, L)] = a16 * vv

                    pltpu.async_copy(wb, wacc_s.at[ib], ssem, add=True)

                    @pl.when(p < n_pairs - 1)
                    def _():
                        start_in(blk + 2, s)
                return carry

            lax.fori_loop(0, n_pairs, pair, 0)
            for s in range(2):
                ib, wb, ssem = stsets[s]
                pltpu.make_async_copy(wb, wacc_s.at[ib], ssem).wait()

        for chunk_i in range(NUM_CHUNKS // NC):
            base = (cid * (NUM_CHUNKS // NC) + chunk_i) * R_CHUNK
            pltpu.sync_copy(w0_hbm.at[pl.ds(base + sid * stripe, stripe)],
                            wacc_s.at[pl.ds(sid * stripe, stripe)])
            plsc.subcore_barrier()
            scan_chunk(base)
            plsc.subcore_barrier()
            pltpu.sync_copy(wacc_s.at[pl.ds(sid * stripe, stripe)],
                            w_out.at[pl.ds(base + sid * stripe, stripe)])
            plsc.subcore_barrier()

        wid = sid * NC + cid
        for q in range(2):
            bstart = wid * eb + q * eb2
            pltpu.sync_copy(brows_hbm.at[pl.ds(bstart, eb2)], brows_v)
            pltpu.sync_copy(bcols_hbm.at[pl.ds(bstart, eb2)], bcols_v)
            pltpu.sync_copy(bvals_hbm.at[pl.ds(bstart, eb2)], bvals_v)

            def bgrp(g, c):
                off = g * L
                r16 = brows_v[pl.ds(off, L)]
                c16 = bcols_v[pl.ds(off, L)]
                a16 = bvals_v[pl.ds(off, L)]
                vv = plsc.load_gather(v_v, [c16])
                bidx_v[pl.ds(off, L)] = r16
                bval_v[pl.ds(off, L)] = a16 * vv
                return c

            lax.fori_loop(0, eb2 // L, bgrp, 0)
            pltpu.sync_copy(bval_v, bacc_s.at[bidx_v], add=True)
        plsc.subcore_barrier()

        @pl.when(sid == 0)
        def _():
            pltpu.sync_copy(bacc_s, pb_out.at[cid])

    return sc_kernel


def _mm_block(x_ref, w_ref, b0_ref, pb_ref, o_ref):
    b = b0_ref[0] + pb_ref[0] + pb_ref[1]
    o_ref[...] = lax.dot_general(
        x_ref[...].astype(jnp.bfloat16), w_ref[...].astype(jnp.bfloat16),
        (((1,), (1,)), ((), ())),
        preferred_element_type=jnp.float32) + b[None, :]


def _matmul(x, w, b0, pb):
    n_tok, d = x.shape
    bm, bn = 1024, 1024
    grid = (d // bn, n_tok // bm)
    return pl.pallas_call(
        _mm_block,
        grid=grid,
        in_specs=[
            pl.BlockSpec((bm, d), lambda j, i: (i, 0)),
            pl.BlockSpec((bn, d), lambda j, i: (j, 0)),
            pl.BlockSpec((1, bn), lambda j, i: (0, j)),
            pl.BlockSpec((NC, bn), lambda j, i: (0, j)),
        ],
        out_specs=pl.BlockSpec((bm, bn), lambda j, i: (i, j)),
        out_shape=jax.ShapeDtypeStruct((n_tok, d), jnp.float32),
    )(x, w, b0, pb)


def kernel(x, V, W0, b0, rows_w, cols_w, vals_w, rows_b, cols_b, vals_b):
    v = V[:, 0]
    nnz_w = rows_w.shape[0]
    nnz_b = rows_b.shape[0]

    nw = NC * NS
    eb = -(-nnz_b // (nw * 2 * L)) * 2 * L
    pad = nw * eb - nnz_b
    rbp = jnp.concatenate([rows_b, jnp.zeros((pad,), rows_b.dtype)])
    cbp = jnp.concatenate([cols_b, jnp.zeros((pad,), cols_b.dtype)])
    vbp = jnp.concatenate([vals_b, jnp.zeros((pad,), vals_b.dtype)])

    sc = _sc_scatter_build(nnz_w, eb)
    w_full, pb = sc(v, W0.reshape(-1), rows_w, cols_w, vals_w,
                    rbp, cbp, vbp, jnp.zeros((D_MODEL,), jnp.float32))
    return _matmul(x, w_full.reshape(D_MODEL, D_MODEL), b0.reshape(1, -1), pb)
